# Initial kernel scaffold; baseline (speedup 1.0000x reference)
import jax
import jax.numpy as jnp
from jax.experimental import pallas as pl


def kernel(x, codebook):
    raise NotImplementedError("write your pallas kernel here")


# speedup vs baseline: 1.1404x; 1.1404x over previous
import jax
import jax.numpy as jnp
from jax.experimental import pallas as pl
from jax.experimental.pallas import tpu as pltpu
from jax.experimental.pallas import tpu_sc as plsc

_BLK = 1024
_W = 128


def _vq_tc_body(x_ref, cb_ref, idx_ref, loss_ref):
    i = pl.program_id(0)
    xb = x_ref[...]
    cb = cb_ref[...]
    mm = jax.lax.dot_general(
        xb, cb, (((1,), (1,)), ((), ())),
        preferred_element_type=jnp.float32)
    xsq = jnp.sum(xb * xb, axis=1, keepdims=True)
    cbsq = jnp.sum(cb * cb, axis=1)
    d = (xsq - 2.0 * mm) + cbsq[None, :]
    idx_ref[0, 0, :] = jnp.argmin(d, axis=1).astype(jnp.int32)
    part = jnp.sum(jnp.min(d, axis=1)).reshape(1, 1)

    @pl.when(i == 0)
    def _init():
        loss_ref[...] = part

    @pl.when(i != 0)
    def _acc():
        loss_ref[...] += part


def _argmin_loss(flat_x, codebook, *, interpret=False):
    n, h = flat_x.shape
    grid = n // _BLK
    return pl.pallas_call(
        _vq_tc_body,
        grid=(grid,),
        in_specs=[
            pl.BlockSpec((_BLK, h), lambda i: (i, 0)),
            pl.BlockSpec(codebook.shape, lambda i: (0, 0)),
        ],
        out_specs=[
            pl.BlockSpec((1, 1, _BLK), lambda i: (i, 0, 0)),
            pl.BlockSpec((1, 1), lambda i: (0, 0)),
        ],
        out_shape=[
            jax.ShapeDtypeStruct((grid, 1, _BLK), jnp.int32),
            jax.ShapeDtypeStruct((1, 1), jnp.float32),
        ],
        interpret=interpret,
    )(flat_x, codebook)


def _gather_rows(codebook, idx):
    n = idx.shape[0]
    k = codebook.shape[0]
    h = 128
    codebook = jnp.concatenate(
        [codebook, jnp.zeros((k, h - codebook.shape[1]), codebook.dtype)],
        axis=1)
    idx2 = idx.reshape(1, n)
    mesh = plsc.VectorSubcoreMesh(core_axis_name="core",
                                  subcore_axis_name="subcore")

    @pl.kernel(out_type=jax.ShapeDtypeStruct((n, h), codebook.dtype),
               mesh=mesh)
    def _sc_gather(cb_hbm, i_hbm, o_hbm):
        def body(i_vmem, o_vmem):
            pltpu.sync_copy(cb_hbm.at[i_vmem.at[0]], o_vmem)

        pltpu.emit_pipeline(
            body,
            grid=(n // _W,),
            in_specs=[pl.BlockSpec((1, _W), index_map=lambda i: (0, i))],
            out_specs=[pl.BlockSpec((_W, h), index_map=lambda i: (i, 0))],
            core_axis_name=("core", "subcore"),
            dimension_semantics=(pltpu.PARALLEL,),
        )(i_hbm, o_hbm)

    return _sc_gather(codebook, idx2)


def kernel(x, codebook):
    b, m, h = x.shape
    flat = x.reshape(-1, h)
    idx3, loss_acc = _argmin_loss(flat, codebook)
    idx = idx3.reshape(-1)
    quantized = _gather_rows(codebook, idx)[:, :h].reshape(b, m, h)
    loss = loss_acc[0, 0] * (1.25 / flat.size)
    return quantized, loss, idx
